# Initial kernel scaffold; baseline (speedup 1.0000x reference)
import jax
import jax.numpy as jnp
from jax.experimental import pallas as pl


def kernel(x, edge_index, batch, W0, b0, g0, be0, W1, b1, g1, be1, W2, b2, g2, be2, W3, b3, g3, be3, fW1, fb1, fW2, fb2, fW3, fb3):
    raise NotImplementedError("write your pallas kernel here")


# speedup vs baseline: 12.0634x; 12.0634x over previous
import functools

import jax
import jax.numpy as jnp
from jax import lax
from jax.experimental import pallas as pl
from jax.experimental.pallas import tpu as pltpu
from jax.experimental.pallas import tpu_sc as plsc

NC = 2
NS = 16
NW = NC * NS
L = 16

N = 10000
NP = 10240
H = 128
G = 128
GP = 144
EPS = 1e-5
ET = N + 320000
EK = -(-ET // (NW * 128))
EP = EK * NW * 128
RB = 10
BR = NP // RB

def _sc_mesh():
    return plsc.VectorSubcoreMesh(
        core_axis_name="c", subcore_axis_name="s",
        num_cores=NC, num_subcores=NS)


def _sc_deg(dst_flat):
    @functools.partial(
        pl.kernel,
        out_type=jax.ShapeDtypeStruct((NW, NP), jnp.float32),
        mesh=_sc_mesh(),
        compiler_params=pltpu.CompilerParams(needs_layout_passes=False),
        scratch_types=[
            pltpu.VMEM((EK * 128,), jnp.int32),
            pltpu.VMEM((NP,), jnp.float32),
        ],
    )
    def k(dst_hbm, out_hbm, idx_v, hist):
        c = lax.axis_index("c")
        s = lax.axis_index("s")
        w = c * NS + s
        pltpu.sync_copy(dst_hbm.at[c, s], idx_v)
        zeros16 = jnp.zeros((L,), jnp.float32)
        ones16 = jnp.ones((L,), jnp.float32)

        @pl.loop(0, NP // L)
        def _z(r):
            hist[pl.ds(r * L, L)] = zeros16

        @pl.loop(0, EK * 128 // L)
        def _h(j):
            v = idx_v[pl.ds(j * L, L)]
            plsc.addupdate_scatter(hist, [v], ones16)

        pltpu.sync_copy(hist, out_hbm.at[w])

    return k(dst_flat)


def _sc_edge(hs, src4, dst4):
    rows_per_tile = NP // NS

    @functools.partial(
        pl.kernel,
        out_type=jax.ShapeDtypeStruct((NC, NP, H), jnp.float32),
        mesh=_sc_mesh(),
        compiler_params=pltpu.CompilerParams(needs_layout_passes=False),
        scratch_types=[
            pltpu.VMEM((EK, 128), jnp.int32),
            pltpu.VMEM((EK, 128), jnp.int32),
            pltpu.VMEM((128, H), jnp.float32),
            pltpu.VMEM_SHARED((NP, H), jnp.float32),
            pltpu.SemaphoreType.DMA,
        ],
    )
    def k(hs_hbm, src_hbm, dst_hbm, out_hbm, idx_s, idx_d, buf, agg_sh, gsem):
        c = lax.axis_index("c")
        s = lax.axis_index("s")
        zeros16 = jnp.zeros((L,), jnp.float32)

        @pl.loop(0, 128)
        def _z(r):
            for l in range(H // L):
                buf[r, pl.ds(l * L, L)] = zeros16

        base = s * rows_per_tile
        for t in range(rows_per_tile // 128):
            pltpu.sync_copy(buf, agg_sh.at[pl.ds(base + t * 128, 128)])
        plsc.subcore_barrier()

        pltpu.sync_copy(src_hbm.at[c, s], idx_s)
        pltpu.sync_copy(dst_hbm.at[c, s], idx_d)

        @pl.loop(0, EK)
        def _e(j):
            pltpu.async_copy(hs_hbm.at[idx_s.at[j]], buf, gsem).wait()
            pltpu.sync_copy(buf, agg_sh.at[idx_d.at[j]], add=True)

        plsc.subcore_barrier()
        pltpu.sync_copy(agg_sh.at[pl.ds(base, rows_per_tile)],
                        out_hbm.at[c, pl.ds(base, rows_per_tile)])

    return k(hs, src4, dst4)


def _sc_pool(h4, batchp):
    rows = NP // NW

    @functools.partial(
        pl.kernel,
        out_type=[
            jax.ShapeDtypeStruct((NW, GP, H), jnp.float32),
            jax.ShapeDtypeStruct((NW, GP, H), jnp.float32),
            jax.ShapeDtypeStruct((NW, GP), jnp.float32),
        ],
        mesh=_sc_mesh(),
        compiler_params=pltpu.CompilerParams(needs_layout_passes=False),
        scratch_types=[
            pltpu.VMEM((rows, H), jnp.float32),
            pltpu.VMEM((rows,), jnp.int32),
            pltpu.VMEM((GP, H), jnp.float32),
            pltpu.VMEM((GP, H), jnp.float32),
            pltpu.VMEM((GP,), jnp.float32),
        ],
    )
    def k(h4_hbm, b_hbm, sums_o, maxs_o, cnts_o, rowbuf, bidx, accs, accm, accc):
        c = lax.axis_index("c")
        s = lax.axis_index("s")
        w = c * NS + s
        start = w * rows
        pltpu.sync_copy(h4_hbm.at[pl.ds(start, rows)], rowbuf)
        pltpu.sync_copy(b_hbm.at[pl.ds(start, rows)], bidx)

        zeros16 = jnp.zeros((L,), jnp.float32)
        iota16 = lax.iota(jnp.int32, L)
        lane0 = iota16 == 0
        one16 = jnp.ones((L,), jnp.float32)

        @pl.loop(0, GP)
        def _z(r):
            for l in range(H // L):
                accs[r, pl.ds(l * L, L)] = zeros16
                accm[r, pl.ds(l * L, L)] = zeros16

        @pl.loop(0, GP // L)
        def _zc(r):
            accc[pl.ds(r * L, L)] = zeros16

        @pl.loop(0, rows)
        def _n(i):
            ivec = jnp.zeros((L,), jnp.int32) + i
            bvec = plsc.load_gather(bidx, [ivec])
            for kk in range(H // L):
                col = iota16 + kk * L
                row = rowbuf[i, pl.ds(kk * L, L)]
                plsc.addupdate_scatter(accs, [bvec, col], row)
                cur = plsc.load_gather(accm, [bvec, col])
                plsc.store_scatter(accm, [bvec, col], jnp.maximum(cur, row))
            plsc.addupdate_scatter(accc, [bvec], one16, mask=lane0)

        pltpu.sync_copy(accs, sums_o.at[w])
        pltpu.sync_copy(accm, maxs_o.at[w])
        pltpu.sync_copy(accc, cnts_o.at[w])

    return k(h4, batchp)


def _dis_from_deg(degp_blk):
    deg = jnp.sum(degp_blk, axis=0)
    return jnp.where(deg > 0, lax.rsqrt(jnp.maximum(deg, 1e-12)), 0.0)


def _tc_prep(xp, W0, degp):
    def body(x_ref, w_ref, deg_ref, o_ref):
        dis = _dis_from_deg(deg_ref[...])
        h = jnp.dot(x_ref[...], w_ref[...], preferred_element_type=jnp.float32)
        o_ref[...] = h * dis[:, None]

    return pl.pallas_call(
        body,
        grid=(RB,),
        in_specs=[
            pl.BlockSpec((BR, H), lambda j: (j, 0)),
            pl.BlockSpec((H, H), lambda j: (0, 0)),
            pl.BlockSpec((NW, BR), lambda j: (0, j)),
        ],
        out_specs=pl.BlockSpec((BR, H), lambda j: (j, 0)),
        out_shape=jax.ShapeDtypeStruct((NP, H), jnp.float32),
    )(xp, W0, degp)


def _tc_stats(aggp, degp):
    def body(a_ref, deg_ref, y_ref, st_ref):
        j = pl.program_id(0)
        dis = _dis_from_deg(deg_ref[...])
        y = (a_ref[0] + a_ref[1]) * dis[:, None]
        rowid = j * BR + lax.broadcasted_iota(jnp.int32, (BR, 1), 0)
        y = jnp.where(rowid < N, y, 0.0)
        y_ref[...] = y
        ssum = jnp.sum(y, axis=0)
        sq = jnp.sum(y * y, axis=0)
        upd = jnp.concatenate(
            [ssum[None], sq[None], jnp.zeros((6, H), jnp.float32)], axis=0)
        st_ref[...] = jnp.where(j == 0, upd, st_ref[...] + upd)

    return pl.pallas_call(
        body,
        grid=(RB,),
        in_specs=[
            pl.BlockSpec((NC, BR, H), lambda j: (0, j, 0)),
            pl.BlockSpec((NW, BR), lambda j: (0, j)),
        ],
        out_specs=[
            pl.BlockSpec((BR, H), lambda j: (j, 0)),
            pl.BlockSpec((8, H), lambda j: (0, 0)),
        ],
        out_shape=[
            jax.ShapeDtypeStruct((NP, H), jnp.float32),
            jax.ShapeDtypeStruct((8, H), jnp.float32),
        ],
    )(aggp, degp)


def _bn_relu(y, st, g_ref, be_ref):
    m = st[0:1] / float(N)
    v = st[1:2] / float(N) - m * m
    return jax.nn.relu((y - m) * lax.rsqrt(v + EPS) * g_ref[...] + be_ref[...])


def _tc_apply_mm(y, stats, g2, be2, Wn, degp):
    def body(y_ref, st_ref, g_ref, be_ref, w_ref, deg_ref, o_ref):
        dis = _dis_from_deg(deg_ref[...])
        hb = _bn_relu(y_ref[...], st_ref[...], g_ref, be_ref)
        h = jnp.dot(hb, w_ref[...], preferred_element_type=jnp.float32)
        o_ref[...] = h * dis[:, None]

    return pl.pallas_call(
        body,
        grid=(RB,),
        in_specs=[
            pl.BlockSpec((BR, H), lambda j: (j, 0)),
            pl.BlockSpec((8, H), lambda j: (0, 0)),
            pl.BlockSpec((1, H), lambda j: (0, 0)),
            pl.BlockSpec((1, H), lambda j: (0, 0)),
            pl.BlockSpec((H, H), lambda j: (0, 0)),
            pl.BlockSpec((NW, BR), lambda j: (0, j)),
        ],
        out_specs=pl.BlockSpec((BR, H), lambda j: (j, 0)),
        out_shape=jax.ShapeDtypeStruct((NP, H), jnp.float32),
    )(y, stats, g2, be2, Wn, degp)


def _tc_apply(y, stats, g2, be2):
    def body(y_ref, st_ref, g_ref, be_ref, o_ref):
        o_ref[...] = _bn_relu(y_ref[...], st_ref[...], g_ref, be_ref)

    return pl.pallas_call(
        body,
        grid=(RB,),
        in_specs=[
            pl.BlockSpec((BR, H), lambda j: (j, 0)),
            pl.BlockSpec((8, H), lambda j: (0, 0)),
            pl.BlockSpec((1, H), lambda j: (0, 0)),
            pl.BlockSpec((1, H), lambda j: (0, 0)),
        ],
        out_specs=pl.BlockSpec((BR, H), lambda j: (j, 0)),
        out_shape=jax.ShapeDtypeStruct((NP, H), jnp.float32),
    )(y, stats, g2, be2)


def _tc_head(sums, maxs, cnts, fW1, fb1, fW2p, fb2p, fW3p, fb3p):
    def body(s_ref, m_ref, c_ref, w1, b1, w2, b2, w3, b3, o_ref):
        s = jnp.sum(s_ref[...], axis=0)[:G]
        mx = jnp.max(m_ref[...], axis=0)[:G]
        cnt = jnp.sum(c_ref[...], axis=0)[:G]
        mean = s / jnp.maximum(cnt, 1.0)[:, None]
        mx = jnp.where(cnt[:, None] > 0, mx, 0.0)
        z = jnp.concatenate([mean, mx, s], axis=1)
        z = jax.nn.relu(
            jnp.dot(z, w1[...], preferred_element_type=jnp.float32) + b1[...])
        z = jax.nn.relu(
            jnp.dot(z, w2[...], preferred_element_type=jnp.float32) + b2[...])
        o_ref[...] = (
            jnp.dot(z, w3[...], preferred_element_type=jnp.float32) + b3[...])

    return pl.pallas_call(
        body,
        out_shape=jax.ShapeDtypeStruct((G, H), jnp.float32),
    )(sums, maxs, cnts, fW1, fb1, fW2p, fb2p, fW3p, fb3p)


def kernel(x, edge_index, batch, W0, b0, g0, be0, W1, b1, g1, be1,
           W2, b2, g2, be2, W3, b3, g3, be3, fW1, fb1, fW2, fb2, fW3, fb3):
    i32 = jnp.int32
    ar = jnp.arange(N, dtype=i32)
    src = jnp.concatenate([edge_index[0].astype(i32), ar])
    dst = jnp.concatenate([edge_index[1].astype(i32), ar])
    padn = EP - ET
    src = jnp.concatenate([src, jnp.zeros((padn,), i32)])
    dst = jnp.concatenate([dst, jnp.full((padn,), N, i32)])
    src4 = src.reshape(NC, NS, EK, 128)
    dst4 = dst.reshape(NC, NS, EK, 128)
    dstf = dst.reshape(NC, NS, EK * 128)

    xp = jnp.pad(x, ((0, NP - N), (0, 0)))
    batchp = jnp.pad(batch.astype(i32), (0, NP - N), constant_values=G)

    gs = [g0, g1, g2, g3]
    bes = [be0, be1, be2, be3]
    Ws = [W0, W1, W2, W3]

    degp = _sc_deg(dstf)
    hs = _tc_prep(xp, W0, degp)
    h4 = None
    for i in range(4):
        aggp = _sc_edge(hs, src4, dst4)
        y, stats = _tc_stats(aggp, degp)
        gi = gs[i].reshape(1, H)
        bei = bes[i].reshape(1, H)
        if i < 3:
            hs = _tc_apply_mm(y, stats, gi, bei, Ws[i + 1], degp)
        else:
            h4 = _tc_apply(y, stats, gi, bei)

    sums, maxs, cnts = _sc_pool(h4, batchp)

    fb1r = fb1.reshape(1, H)
    fW2p = jnp.pad(fW2, ((0, 0), (0, H - fW2.shape[1])))
    fb2p = jnp.pad(fb2, (0, H - fb2.shape[0])).reshape(1, H)
    fW3p = jnp.pad(fW3, ((0, H - fW3.shape[0]), (0, H - fW3.shape[1])))
    fb3p = jnp.pad(fb3, (0, H - fb3.shape[0])).reshape(1, H)

    out = _tc_head(sums, maxs, cnts, fW1, fb1r, fW2p, fb2p, fW3p, fb3p)
    return out[:, :1]
